# baseline (speedup 1.0000x reference)
import jax
import jax.numpy as jnp
from jax import lax
from jax.experimental import pallas as pl
from jax.experimental.pallas import tpu as pltpu
from jax.experimental.pallas import tpu_sc as plsc

D_MODEL = 768
D_FF = 2048
NE = 64
T = 2048
CAP = 80
NSLOT = NE * CAP
NBUF = NSLOT + CAP
DUMP = NSLOT
GW = 128
SLW = 8
RB = 256
NRB = T // RB


def _router_body(x_ref, wg_ref, slot_ref, w_ref, aux_ref, base_ref, me_ref):
    i = pl.program_id(0)

    @pl.when(i == 0)
    def _init():
        base_ref[...] = jnp.zeros_like(base_ref)
        me_ref[...] = jnp.zeros_like(me_ref)

    x = x_ref[...]
    logits = jnp.dot(x, wg_ref[...], preferred_element_type=jnp.float32)
    m = jnp.max(logits, axis=1, keepdims=True)
    ex = jnp.exp(logits - m)
    s = jnp.sum(ex, axis=1, keepdims=True)
    gates = ex / s
    gmax = jnp.max(gates, axis=1, keepdims=True)
    iota_e = lax.broadcasted_iota(jnp.int32, (RB, NE), 1).astype(jnp.float32)
    eidf = jnp.min(jnp.where(gates == gmax, iota_e, jnp.float32(NE)),
                   axis=1, keepdims=True)
    mask = (iota_e == eidf).astype(jnp.float32)
    r_i = lax.broadcasted_iota(jnp.int32, (RB, RB), 0)
    c_i = lax.broadcasted_iota(jnp.int32, (RB, RB), 1)
    tri = (r_i >= c_i).astype(jnp.float32)
    incl = jnp.dot(tri, mask, preferred_element_type=jnp.float32)
    base = base_ref[0:1, :]
    posf = jnp.sum(mask * (incl - 1.0 + base), axis=1, keepdims=True)
    base_ref[0:1, :] = base + jnp.sum(mask, axis=0, keepdims=True)
    me_ref[0:1, :] = me_ref[0:1, :] + jnp.sum(gates, axis=0, keepdims=True)

    keep = posf < jnp.float32(CAP)
    slotf = jnp.where(keep, eidf * CAP + jnp.minimum(posf, CAP - 1.0),
                      jnp.float32(DUMP))
    w_eff = jnp.where(keep, gmax, 0.0)
    eye = (r_i == c_i).astype(jnp.float32)
    slotT = lax.dot_general(slotf, eye, (((0,), (0,)), ((), ())),
                            preferred_element_type=jnp.float32)
    slot_ref[0, 0, pl.ds(i * RB, RB)] = slotT[0].astype(jnp.int32)
    w_ref[...] = jnp.broadcast_to(w_eff, (RB, GW))

    @pl.when(i == NRB - 1)
    def _finish():
        fe = base_ref[0:1, :] * (1.0 / T)
        me = me_ref[0:1, :] * (1.0 / T)
        aux_ref[0, 0] = jnp.float32(NE) * jnp.sum(fe * me)


def _make_router(interpret=False):
    return pl.pallas_call(
        _router_body,
        grid=(NRB,),
        in_specs=[
            pl.BlockSpec((RB, D_MODEL), lambda i: (i, 0)),
            pl.BlockSpec((D_MODEL, NE), lambda i: (0, 0)),
        ],
        out_specs=[
            pl.BlockSpec((1, 1, T), lambda i: (0, 0, 0)),
            pl.BlockSpec((RB, GW), lambda i: (i, 0)),
            pl.BlockSpec(memory_space=pltpu.SMEM),
        ],
        out_shape=[
            jax.ShapeDtypeStruct((1, 1, T), jnp.int32),
            jax.ShapeDtypeStruct((T, GW), jnp.float32),
            jax.ShapeDtypeStruct((1, 1), jnp.float32),
        ],
        scratch_shapes=[
            pltpu.VMEM((8, NE), jnp.float32),
            pltpu.VMEM((8, NE), jnp.float32),
        ],
        interpret=interpret,
    )


_NC = 2
_NS = 16
_NW = _NC * _NS
CHUNK = T // _NW


H = CHUNK // 2


def _dispatch_body(x_hbm, slot1d_hbm, w_hbm, buf_hbm, gtab_hbm,
                   wv_a, wv_b, idx_a, idx_b, rows_a, rows_b,
                   sxa, sxb, ssa, ssb, sga, sgb):
    wid = lax.axis_index("s") * _NC + lax.axis_index("c")
    base = wid * CHUNK
    cxa = pltpu.async_copy(x_hbm.at[pl.ds(base, H)], rows_a, sxa)
    cxb = pltpu.async_copy(x_hbm.at[pl.ds(base + H, H)], rows_b, sxb)
    pltpu.sync_copy(slot1d_hbm.at[pl.ds(base, H)], idx_a)
    pltpu.sync_copy(slot1d_hbm.at[pl.ds(base + H, H)], idx_b)
    pltpu.sync_copy(w_hbm.at[pl.ds(base, H)], wv_a)
    pltpu.sync_copy(w_hbm.at[pl.ds(base + H, H)], wv_b)
    cga = pltpu.async_copy(wv_a, gtab_hbm.at[idx_a], sga)
    cgb = pltpu.async_copy(wv_b, gtab_hbm.at[idx_b], sgb)
    cxa.wait()
    ca = pltpu.async_copy(rows_a, buf_hbm.at[idx_a], ssa)
    cxb.wait()
    cb = pltpu.async_copy(rows_b, buf_hbm.at[idx_b], ssb)
    cga.wait()
    cgb.wait()
    ca.wait()
    cb.wait()


def _combine_body(y_hbm, slot1d_hbm, out_hbm, idx_a, idx_b, rows_a, rows_b,
                  sga, sgb, swa, swb):
    wid = lax.axis_index("s") * _NC + lax.axis_index("c")
    base = wid * CHUNK
    pltpu.sync_copy(slot1d_hbm.at[pl.ds(base, H)], idx_a)
    pltpu.sync_copy(slot1d_hbm.at[pl.ds(base + H, H)], idx_b)
    ga = pltpu.async_copy(y_hbm.at[idx_a], rows_a, sga)
    gb = pltpu.async_copy(y_hbm.at[idx_b], rows_b, sgb)
    ga.wait()
    wa = pltpu.async_copy(rows_a, out_hbm.at[pl.ds(base, H)], swa)
    gb.wait()
    wb = pltpu.async_copy(rows_b, out_hbm.at[pl.ds(base + H, H)], swb)
    wa.wait()
    wb.wait()


def _sc_mesh():
    return plsc.VectorSubcoreMesh(core_axis_name="c", subcore_axis_name="s",
                                  num_cores=_NC, num_subcores=_NS)


def _make_dispatch(interpret=False):
    return pl.kernel(
        _dispatch_body,
        out_type=[
            jax.ShapeDtypeStruct((NBUF, D_MODEL), jnp.float32),
            jax.ShapeDtypeStruct((NBUF, GW), jnp.float32),
        ],
        mesh=_sc_mesh(),
        scratch_types=[
            pltpu.VMEM((H, GW), jnp.float32),
            pltpu.VMEM((H, GW), jnp.float32),
            pltpu.VMEM((H,), jnp.int32),
            pltpu.VMEM((H,), jnp.int32),
            pltpu.VMEM((H, D_MODEL), jnp.float32),
            pltpu.VMEM((H, D_MODEL), jnp.float32),
            pltpu.SemaphoreType.DMA,
            pltpu.SemaphoreType.DMA,
            pltpu.SemaphoreType.DMA,
            pltpu.SemaphoreType.DMA,
            pltpu.SemaphoreType.DMA,
            pltpu.SemaphoreType.DMA,
        ],
        interpret=interpret,
    )


def _make_combine(interpret=False):
    return pl.kernel(
        _combine_body,
        out_type=jax.ShapeDtypeStruct((T, D_MODEL), jnp.float32),
        mesh=_sc_mesh(),
        scratch_types=[
            pltpu.VMEM((H,), jnp.int32),
            pltpu.VMEM((H,), jnp.int32),
            pltpu.VMEM((H, D_MODEL), jnp.float32),
            pltpu.VMEM((H, D_MODEL), jnp.float32),
            pltpu.SemaphoreType.DMA,
            pltpu.SemaphoreType.DMA,
            pltpu.SemaphoreType.DMA,
            pltpu.SemaphoreType.DMA,
        ],
        interpret=interpret,
    )


NF = 1
FB = D_FF // NF


def _ffn_body(buf_ref, w1_ref, b1_ref, w2_ref, b2_ref, g_ref, y_ref):
    e = pl.program_id(0)
    f = pl.program_id(1)
    xb = jnp.where(e < NE, buf_ref[...], 0.0)
    h = jnp.dot(xb, w1_ref[0], preferred_element_type=jnp.float32)
    h = jnp.maximum(h + b1_ref[0], 0.0)
    part = jnp.dot(h, w2_ref[0], preferred_element_type=jnp.float32)

    if NF > 1:
        @pl.when(f < NF - 1)
        def _partial():
            y_ref[...] = jnp.where(f == 0, part, y_ref[...] + part)

    @pl.when(f == NF - 1)
    def _last():
        acc = part if NF == 1 else y_ref[...] + part
        g2d = g_ref[0, :, 0:1]
        y_ref[...] = (acc + b2_ref[0]) * g2d


def _make_ffn(interpret=False):
    emap = lambda e, f: (jnp.minimum(e, NE - 1), 0, 0)
    return pl.pallas_call(
        _ffn_body,
        grid=(NBUF // CAP, NF),
        in_specs=[
            pl.BlockSpec((CAP, D_MODEL), lambda e, f: (e, 0)),
            pl.BlockSpec((1, D_MODEL, FB),
                         lambda e, f: (jnp.minimum(e, NE - 1), 0, f)),
            pl.BlockSpec((1, 1, FB),
                         lambda e, f: (jnp.minimum(e, NE - 1), 0, f)),
            pl.BlockSpec((1, FB, D_MODEL),
                         lambda e, f: (jnp.minimum(e, NE - 1), f, 0)),
            pl.BlockSpec((1, 1, D_MODEL), emap),
            pl.BlockSpec((1, CAP, GW), lambda e, f: (e, 0, 0)),
        ],
        out_specs=pl.BlockSpec((CAP, D_MODEL), lambda e, f: (e, 0)),
        out_shape=jax.ShapeDtypeStruct((NBUF, D_MODEL), jnp.float32),
        interpret=interpret,
    )


def _moe(x, Wg, W1, b1, W2, b2, interpret=False):
    x2 = x.reshape(T, D_MODEL)
    slots3, weff2, aux = _make_router(interpret)(x2, Wg)
    slots1d = slots3.reshape(T)
    buf, gtab = _make_dispatch(interpret)(x2, slots1d, weff2)
    y = _make_ffn(interpret)(buf, W1, b1.reshape(NE, 1, D_FF),
                             W2, b2.reshape(NE, 1, D_MODEL),
                             gtab.reshape(NBUF // CAP, CAP, GW))
    out = _make_combine(interpret)(y, slots1d)
    return out.reshape(1, T, D_MODEL), aux.reshape(())


def kernel(x, Wg, W1, b1, W2, b2):
    return _moe(x, Wg, W1, b1, W2, b2)


# speedup vs baseline: 1.0120x; 1.0120x over previous
import jax
import jax.numpy as jnp
from jax import lax
from jax.experimental import pallas as pl
from jax.experimental.pallas import tpu as pltpu
from jax.experimental.pallas import tpu_sc as plsc

D_MODEL = 768
D_FF = 2048
NE = 64
T = 2048
CAP = 80
NSLOT = NE * CAP
NBUF = NSLOT + CAP
DUMP = NSLOT
GW = 128
SLW = 8
RB = 256
NRB = T // RB


def _router_body(x_ref, wg_ref, slot_ref, w_ref, aux_ref, base_ref, me_ref):
    i = pl.program_id(0)

    @pl.when(i == 0)
    def _init():
        base_ref[...] = jnp.zeros_like(base_ref)
        me_ref[...] = jnp.zeros_like(me_ref)

    x = x_ref[...]
    logits = jnp.dot(x, wg_ref[...], preferred_element_type=jnp.float32)
    m = jnp.max(logits, axis=1, keepdims=True)
    ex = jnp.exp(logits - m)
    s = jnp.sum(ex, axis=1, keepdims=True)
    gates = ex / s
    gmax = jnp.max(gates, axis=1, keepdims=True)
    iota_e = lax.broadcasted_iota(jnp.int32, (RB, NE), 1).astype(jnp.float32)
    eidf = jnp.min(jnp.where(gates == gmax, iota_e, jnp.float32(NE)),
                   axis=1, keepdims=True)
    mask = (iota_e == eidf).astype(jnp.float32)
    r_i = lax.broadcasted_iota(jnp.int32, (RB, RB), 0)
    c_i = lax.broadcasted_iota(jnp.int32, (RB, RB), 1)
    tri = (r_i >= c_i).astype(jnp.float32)
    incl = jnp.dot(tri, mask, preferred_element_type=jnp.float32)
    base = base_ref[0:1, :]
    posf = jnp.sum(mask * (incl - 1.0 + base), axis=1, keepdims=True)
    base_ref[0:1, :] = base + jnp.sum(mask, axis=0, keepdims=True)
    me_ref[0:1, :] = me_ref[0:1, :] + jnp.sum(gates, axis=0, keepdims=True)

    keep = posf < jnp.float32(CAP)
    slotf = jnp.where(keep, eidf * CAP + jnp.minimum(posf, CAP - 1.0),
                      jnp.float32(DUMP))
    w_eff = jnp.where(keep, gmax, 0.0)
    slot_ref[...] = jnp.broadcast_to(slotf, (RB, SLW)).astype(jnp.int32)
    w_ref[...] = jnp.broadcast_to(w_eff, (RB, GW))

    @pl.when(i == NRB - 1)
    def _finish():
        fe = base_ref[0:1, :] * (1.0 / T)
        me = me_ref[0:1, :] * (1.0 / T)
        aux_ref[0, 0] = jnp.float32(NE) * jnp.sum(fe * me)


def _make_router(interpret=False):
    return pl.pallas_call(
        _router_body,
        grid=(NRB,),
        in_specs=[
            pl.BlockSpec((RB, D_MODEL), lambda i: (i, 0)),
            pl.BlockSpec((D_MODEL, NE), lambda i: (0, 0)),
        ],
        out_specs=[
            pl.BlockSpec((RB, SLW), lambda i: (i, 0)),
            pl.BlockSpec((RB, GW), lambda i: (i, 0)),
            pl.BlockSpec(memory_space=pltpu.SMEM),
        ],
        out_shape=[
            jax.ShapeDtypeStruct((T, SLW), jnp.int32),
            jax.ShapeDtypeStruct((T, GW), jnp.float32),
            jax.ShapeDtypeStruct((1, 1), jnp.float32),
        ],
        scratch_shapes=[
            pltpu.VMEM((8, NE), jnp.float32),
            pltpu.VMEM((8, NE), jnp.float32),
        ],
        interpret=interpret,
    )


_NC = 2
_NS = 16
_NW = _NC * _NS
CHUNK = T // _NW


H = CHUNK // 2


def _dispatch_body(x_hbm, slot1d_hbm, w_hbm, buf_hbm, gtab_hbm,
                   wv_a, wv_b, idx_a, idx_b, rows_a, rows_b,
                   sxa, sxb, ssa, ssb, sga, sgb):
    wid = lax.axis_index("s") * _NC + lax.axis_index("c")
    base = wid * CHUNK
    cxa = pltpu.async_copy(x_hbm.at[pl.ds(base, H)], rows_a, sxa)
    cxb = pltpu.async_copy(x_hbm.at[pl.ds(base + H, H)], rows_b, sxb)
    pltpu.sync_copy(slot1d_hbm.at[pl.ds(base, H)], idx_a)
    pltpu.sync_copy(slot1d_hbm.at[pl.ds(base + H, H)], idx_b)
    pltpu.sync_copy(w_hbm.at[pl.ds(base, H)], wv_a)
    pltpu.sync_copy(w_hbm.at[pl.ds(base + H, H)], wv_b)
    cga = pltpu.async_copy(wv_a, gtab_hbm.at[idx_a], sga)
    cgb = pltpu.async_copy(wv_b, gtab_hbm.at[idx_b], sgb)
    cxa.wait()
    ca = pltpu.async_copy(rows_a, buf_hbm.at[idx_a], ssa)
    cxb.wait()
    cb = pltpu.async_copy(rows_b, buf_hbm.at[idx_b], ssb)
    cga.wait()
    cgb.wait()
    ca.wait()
    cb.wait()


def _combine_body(y_hbm, slot1d_hbm, out_hbm, idx_a, idx_b, rows_a, rows_b,
                  sga, sgb, swa, swb):
    wid = lax.axis_index("s") * _NC + lax.axis_index("c")
    base = wid * CHUNK
    pltpu.sync_copy(slot1d_hbm.at[pl.ds(base, H)], idx_a)
    pltpu.sync_copy(slot1d_hbm.at[pl.ds(base + H, H)], idx_b)
    ga = pltpu.async_copy(y_hbm.at[idx_a], rows_a, sga)
    gb = pltpu.async_copy(y_hbm.at[idx_b], rows_b, sgb)
    ga.wait()
    wa = pltpu.async_copy(rows_a, out_hbm.at[pl.ds(base, H)], swa)
    gb.wait()
    wb = pltpu.async_copy(rows_b, out_hbm.at[pl.ds(base + H, H)], swb)
    wa.wait()
    wb.wait()


def _sc_mesh():
    return plsc.VectorSubcoreMesh(core_axis_name="c", subcore_axis_name="s",
                                  num_cores=_NC, num_subcores=_NS)


def _make_dispatch(interpret=False):
    return pl.kernel(
        _dispatch_body,
        out_type=[
            jax.ShapeDtypeStruct((NBUF, D_MODEL), jnp.float32),
            jax.ShapeDtypeStruct((NBUF, GW), jnp.float32),
        ],
        mesh=_sc_mesh(),
        scratch_types=[
            pltpu.VMEM((H, GW), jnp.float32),
            pltpu.VMEM((H, GW), jnp.float32),
            pltpu.VMEM((H,), jnp.int32),
            pltpu.VMEM((H,), jnp.int32),
            pltpu.VMEM((H, D_MODEL), jnp.float32),
            pltpu.VMEM((H, D_MODEL), jnp.float32),
            pltpu.SemaphoreType.DMA,
            pltpu.SemaphoreType.DMA,
            pltpu.SemaphoreType.DMA,
            pltpu.SemaphoreType.DMA,
            pltpu.SemaphoreType.DMA,
            pltpu.SemaphoreType.DMA,
        ],
        interpret=interpret,
    )


def _make_combine(interpret=False):
    return pl.kernel(
        _combine_body,
        out_type=jax.ShapeDtypeStruct((T, D_MODEL), jnp.float32),
        mesh=_sc_mesh(),
        scratch_types=[
            pltpu.VMEM((H,), jnp.int32),
            pltpu.VMEM((H,), jnp.int32),
            pltpu.VMEM((H, D_MODEL), jnp.float32),
            pltpu.VMEM((H, D_MODEL), jnp.float32),
            pltpu.SemaphoreType.DMA,
            pltpu.SemaphoreType.DMA,
            pltpu.SemaphoreType.DMA,
            pltpu.SemaphoreType.DMA,
        ],
        interpret=interpret,
    )


NF = 1
FB = D_FF // NF


def _ffn_body(buf_ref, w1_ref, b1_ref, w2_ref, b2_ref, g_ref, y_ref):
    e = pl.program_id(0)
    f = pl.program_id(1)
    xb = jnp.where(e < NE, buf_ref[...], 0.0)
    h = jnp.dot(xb, w1_ref[0], preferred_element_type=jnp.float32)
    h = jnp.maximum(h + b1_ref[0], 0.0)
    part = jnp.dot(h, w2_ref[0], preferred_element_type=jnp.float32)

    if NF > 1:
        @pl.when(f < NF - 1)
        def _partial():
            y_ref[...] = jnp.where(f == 0, part, y_ref[...] + part)

    @pl.when(f == NF - 1)
    def _last():
        acc = part if NF == 1 else y_ref[...] + part
        g2d = g_ref[0, :, 0:1]
        y_ref[...] = (acc + b2_ref[0]) * g2d


def _make_ffn(interpret=False):
    emap = lambda e, f: (jnp.minimum(e, NE - 1), 0, 0)
    return pl.pallas_call(
        _ffn_body,
        grid=(NBUF // CAP, NF),
        in_specs=[
            pl.BlockSpec((CAP, D_MODEL), lambda e, f: (e, 0)),
            pl.BlockSpec((1, D_MODEL, FB),
                         lambda e, f: (jnp.minimum(e, NE - 1), 0, f)),
            pl.BlockSpec((1, 1, FB),
                         lambda e, f: (jnp.minimum(e, NE - 1), 0, f)),
            pl.BlockSpec((1, FB, D_MODEL),
                         lambda e, f: (jnp.minimum(e, NE - 1), f, 0)),
            pl.BlockSpec((1, 1, D_MODEL), emap),
            pl.BlockSpec((1, CAP, GW), lambda e, f: (e, 0, 0)),
        ],
        out_specs=pl.BlockSpec((CAP, D_MODEL), lambda e, f: (e, 0)),
        out_shape=jax.ShapeDtypeStruct((NBUF, D_MODEL), jnp.float32),
        interpret=interpret,
    )


def _moe(x, Wg, W1, b1, W2, b2, interpret=False):
    x2 = x.reshape(T, D_MODEL)
    slots2, weff2, aux = _make_router(interpret)(x2, Wg)
    slots1d = slots2[:, 0]
    buf, gtab = _make_dispatch(interpret)(x2, slots1d, weff2)
    y = _make_ffn(interpret)(buf, W1, b1.reshape(NE, 1, D_FF),
                             W2, b2.reshape(NE, 1, D_MODEL),
                             gtab.reshape(NBUF // CAP, CAP, GW))
    out = _make_combine(interpret)(y, slots1d)
    return out.reshape(1, T, D_MODEL), aux.reshape(())


def kernel(x, Wg, W1, b1, W2, b2):
    return _moe(x, Wg, W1, b1, W2, b2)
